# baseline (speedup 1.0000x reference)
import functools

import jax
import jax.numpy as jnp
from jax import lax
from jax.experimental import pallas as pl
from jax.experimental.pallas import tpu as pltpu
from jax.experimental.pallas import tpu_sc as plsc

VOCAB = 100000
D = 64
B = 4096
H = 50
NB = B * H
NC = 2
NS = 16
NW = NC * NS
B_PER_W = NB // NW
G = 640
NG = B_PER_W // G


def _make_kernel():
    mesh = plsc.VectorSubcoreMesh(core_axis_name="c", subcore_axis_name="s")

    nbuf = 2

    @functools.partial(
        pl.kernel,
        mesh=mesh,
        compiler_params=pltpu.CompilerParams(use_tc_tiling_on_sc=False),
        out_type=jax.ShapeDtypeStruct((NW, NG, G, D), jnp.float32),
        scratch_types=[
            pltpu.VMEM((NG, G), jnp.int32),
            pltpu.VMEM((nbuf, G, D), jnp.float32),
        ]
        + [pltpu.SemaphoreType.DMA] * nbuf,
    )
    def embed(idx_hbm, table_hbm, out_hbm, idx_v, rows_v, *gsems):
        wid = lax.axis_index("s") * NC + lax.axis_index("c")
        pltpu.sync_copy(idx_hbm.at[wid], idx_v)

        for b in range(nbuf):
            pltpu.async_copy(table_hbm.at[idx_v.at[b]], rows_v.at[b], gsems[b])

        def body(g, carry):
            for b in range(nbuf):
                j = g * nbuf + b
                pltpu.make_async_copy(
                    table_hbm.at[idx_v.at[0]], rows_v.at[b], gsems[b]
                ).wait()
                pltpu.sync_copy(rows_v.at[b], out_hbm.at[wid, j])
                jn = j + nbuf

                @pl.when(jn < NG)
                def _():
                    pltpu.async_copy(
                        table_hbm.at[idx_v.at[jn]], rows_v.at[b], gsems[b]
                    )

            return carry

        lax.fori_loop(0, NG // nbuf, body, 0)

    return embed


_embed = _make_kernel()


def kernel(batch_data, pretrained_word_embeddings):
    idx = batch_data.astype(jnp.int32).reshape(NW, NG, G)
    out = _embed(idx, pretrained_word_embeddings)
    return out.reshape(B, H, D)


# speedup vs baseline: 1.3881x; 1.3881x over previous
import functools

import jax
import jax.numpy as jnp
from jax import lax
from jax.experimental import pallas as pl
from jax.experimental.layout import Layout, with_layout_constraint
from jax.experimental.pallas import tpu as pltpu
from jax.experimental.pallas import tpu_sc as plsc

VOCAB = 100000
D = 64
B = 4096
H = 50
NB = B * H
NC = 2
NS = 16
NW = NC * NS
B_PER_W = NB // NW
G = 640
NG = B_PER_W // G


def _make_kernel():
    mesh = plsc.VectorSubcoreMesh(core_axis_name="c", subcore_axis_name="s")
    nbuf = 2

    @functools.partial(
        pl.kernel,
        mesh=mesh,
        compiler_params=pltpu.CompilerParams(use_tc_tiling_on_sc=False),
        out_type=jax.ShapeDtypeStruct((NW, NG, G, D), jnp.float32),
        scratch_types=[
            pltpu.VMEM((B_PER_W,), jnp.int32),
            pltpu.VMEM((nbuf, G, D), jnp.float32),
        ]
        + [pltpu.SemaphoreType.DMA] * nbuf,
    )
    def embed(idx_hbm, table_hbm, out_hbm, idx_v, rows_v, *gsems):
        wid = lax.axis_index("s") * NC + lax.axis_index("c")
        pltpu.sync_copy(idx_hbm.at[wid], idx_v)

        for b in range(nbuf):
            pltpu.async_copy(
                table_hbm.at[idx_v.at[pl.ds(b * G, G)]], rows_v.at[b], gsems[b]
            )

        def body(g, carry):
            for b in range(nbuf):
                j = g * nbuf + b
                pltpu.make_async_copy(
                    table_hbm.at[idx_v.at[pl.ds(0, G)]], rows_v.at[b], gsems[b]
                ).wait()
                pltpu.sync_copy(rows_v.at[b], out_hbm.at[wid, j])
                jn = j + nbuf

                @pl.when(jn < NG)
                def _():
                    pltpu.async_copy(
                        table_hbm.at[idx_v.at[pl.ds(jn * G, G)]],
                        rows_v.at[b],
                        gsems[b],
                    )

            return carry

        lax.fori_loop(0, NG // nbuf, body, 0)

    return embed


_embed = _make_kernel()


_TABLE_FMT = Layout((0, 1), ((8, 64),))
_IDX_FMT = Layout((0, 1), ((8, 128),))
_OUT_FMT = Layout((0, 1, 2), ((2, 64),))


def kernel(batch_data, pretrained_word_embeddings):
    table = with_layout_constraint(pretrained_word_embeddings, _TABLE_FMT)
    idx = batch_data.astype(jnp.int32).reshape(NW, B_PER_W)
    out = _embed(idx, table)
    return with_layout_constraint(out.reshape(B, H, D), _OUT_FMT)


# speedup vs baseline: 1.3895x; 1.0010x over previous
import functools

import jax
import jax.numpy as jnp
from jax import lax
from jax.experimental import pallas as pl
from jax.experimental.layout import Layout, with_layout_constraint
from jax.experimental.pallas import tpu as pltpu
from jax.experimental.pallas import tpu_sc as plsc

VOCAB = 100000
D = 64
B = 4096
H = 50
NB = B * H
NC = 2
NS = 16
NW = NC * NS
B_PER_W = NB // NW
G = 640
NG = B_PER_W // G


def _make_kernel():
    mesh = plsc.VectorSubcoreMesh(core_axis_name="c", subcore_axis_name="s")
    nbuf = 2

    @functools.partial(
        pl.kernel,
        mesh=mesh,
        compiler_params=pltpu.CompilerParams(use_tc_tiling_on_sc=False),
        out_type=jax.ShapeDtypeStruct((NW, NG, G, D), jnp.float32),
        scratch_types=[
            pltpu.VMEM((B_PER_W,), jnp.int32),
            pltpu.VMEM((nbuf, G, D), jnp.float32),
        ]
        + [pltpu.SemaphoreType.DMA] * nbuf,
    )
    def embed(idx_hbm, table_hbm, out_hbm, idx_v, rows_v, *gsems):
        wid = lax.axis_index("s") * NC + lax.axis_index("c")
        pltpu.sync_copy(idx_hbm.at[wid], idx_v)

        for b in range(nbuf):
            pltpu.async_copy(
                table_hbm.at[idx_v.at[pl.ds(b * G, G)]], rows_v.at[b], gsems[b]
            )

        def body(g, carry):
            for b in range(nbuf):
                j = g * nbuf + b
                pltpu.make_async_copy(
                    table_hbm.at[idx_v.at[pl.ds(0, G)]], rows_v.at[b], gsems[b]
                ).wait()
                pltpu.sync_copy(rows_v.at[b], out_hbm.at[wid, j])
                jn = j + nbuf

                @pl.when(jn < NG)
                def _():
                    pltpu.async_copy(
                        table_hbm.at[idx_v.at[pl.ds(jn * G, G)]],
                        rows_v.at[b],
                        gsems[b],
                    )

            return carry

        lax.fori_loop(0, NG // nbuf, body, 0)

    return embed


_embed = _make_kernel()


_TABLE_FMT = Layout((0, 1), ((8, 64),))
_IDX_FMT = Layout((0, 1), ((8, 128),))
_OUT_FMT = Layout((0, 1, 2))


def kernel(batch_data, pretrained_word_embeddings):
    table = with_layout_constraint(pretrained_word_embeddings, _TABLE_FMT)
    idx = batch_data.astype(jnp.int32).reshape(NW, B_PER_W)
    out = _embed(idx, table)
    return with_layout_constraint(out.reshape(B, H, D), _OUT_FMT)
